# baseline (speedup 1.0000x reference)
import functools

import jax
import jax.numpy as jnp
from jax import lax
from jax.experimental import pallas as pl
from jax.experimental.pallas import tpu as pltpu
from jax.experimental.pallas import tpu_sc as plsc

N = 50000
E = 1600000
F_IN = 50
F_HID = 16

N_PAD = 50176
EDGES_PER_CORE = E // 2
CHUNK = 4096
FULL_CHUNKS = EDGES_PER_CORE // CHUNK
TAIL = EDGES_PER_CORE - FULL_CHUNKS * CHUNK
E_PAD = 1605632
DEG_ROWS_PER_TILE = E_PAD // 128 // 32
DEG_CHUNK_ROWS = 8
NCH_D = DEG_ROWS_PER_TILE // DEG_CHUNK_ROWS
N_SLICE = N_PAD // 16

NBLK = 1024
N_GRID = N_PAD // NBLK

_sc_mesh = plsc.VectorSubcoreMesh(core_axis_name="c", subcore_axis_name="s")


@functools.partial(
    pl.kernel,
    out_type=jax.ShapeDtypeStruct((2 * N_PAD,), jnp.float32),
    mesh=_sc_mesh,
    compiler_params=pltpu.CompilerParams(needs_layout_passes=False),
    scratch_types=[
        pltpu.VMEM((DEG_CHUNK_ROWS, 128), jnp.int32),
        pltpu.VMEM((DEG_CHUNK_ROWS, 128), jnp.int32),
        pltpu.VMEM((DEG_CHUNK_ROWS, 128), jnp.float32),
        pltpu.VMEM((DEG_CHUNK_ROWS, 128), jnp.float32),
        pltpu.VMEM((N_SLICE,), jnp.float32),
        pltpu.VMEM_SHARED((N_PAD,), jnp.float32),
        pltpu.SemaphoreType.DMA,
        pltpu.SemaphoreType.DMA,
        pltpu.SemaphoreType.DMA,
    ],
)
def _deg_kernel(dst_hbm, w_hbm, deg_hbm, dst0, dst1, w0, w1, z_v, deg_sh,
                sem0, sem1, sem_sc):
    c = lax.axis_index("c")
    s = lax.axis_index("s")

    def zb(i, carry):
        z_v[pl.ds(i * 16, 16)] = jnp.zeros((16,), jnp.float32)
        return carry

    lax.fori_loop(0, N_SLICE // 16, zb, 0, unroll=8)
    pltpu.sync_copy(z_v, deg_sh.at[pl.ds(s * N_SLICE, N_SLICE)])
    plsc.subcore_barrier()

    row_base = (c * 16 + s) * DEG_ROWS_PER_TILE
    dsts, ws, sems = (dst0, dst1), (w0, w1), (sem0, sem1)

    def start(k, b):
        r0 = row_base + k * DEG_CHUNK_ROWS
        pltpu.async_copy(dst_hbm.at[pl.ds(r0, DEG_CHUNK_ROWS)], dsts[b],
                         sems[b])
        pltpu.async_copy(w_hbm.at[pl.ds(r0, DEG_CHUNK_ROWS)], ws[b],
                         sems[b])

    def wait_in(b):
        pltpu.make_async_copy(dst_hbm.at[pl.ds(0, DEG_CHUNK_ROWS)], dsts[b],
                              sems[b]).wait()
        pltpu.make_async_copy(w_hbm.at[pl.ds(0, DEG_CHUNK_ROWS)], ws[b],
                              sems[b]).wait()

    def process(b):
        dst_v, w_v = dsts[b], ws[b]
        for j in range(DEG_CHUNK_ROWS):
            pltpu.async_copy(w_v.at[j], deg_sh.at[dst_v.at[j]], sem_sc,
                             add=True)
        pltpu.make_async_copy(w_hbm.at[pl.ds(0, DEG_CHUNK_ROWS)], w_v,
                              sem_sc).wait()

    start(0, 0)

    def outer(g, carry):
        for b in range(2):
            k = g * 2 + b

            @pl.when(k + 1 < NCH_D)
            def _():
                start(k + 1, 1 - b)

            wait_in(b)
            process(b)
        return carry

    lax.fori_loop(0, NCH_D // 2, outer, 0)
    wait_in(0)
    process(0)

    plsc.subcore_barrier()
    pltpu.sync_copy(deg_sh.at[pl.ds(s * N_SLICE, N_SLICE)], z_v)
    pltpu.sync_copy(z_v, deg_hbm.at[pl.ds(c * N_PAD + s * N_SLICE, N_SLICE)])


@functools.partial(
    pl.kernel,
    out_type=jax.ShapeDtypeStruct((2 * F_HID * N_PAD,), jnp.float32),
    mesh=_sc_mesh,
    compiler_params=pltpu.CompilerParams(needs_layout_passes=False),
    scratch_types=[
        pltpu.VMEM((N_PAD,), jnp.float32),
        pltpu.VMEM((N_PAD,), jnp.float32),
        pltpu.VMEM((CHUNK,), jnp.int32),
        pltpu.VMEM((CHUNK,), jnp.int32),
        pltpu.VMEM((CHUNK,), jnp.float32),
        pltpu.VMEM((CHUNK,), jnp.float32),
        pltpu.SemaphoreType.DMA,
        pltpu.SemaphoreType.DMA,
    ],
)
def _agg_kernel(gT_hbm, pk_hbm, w_hbm, out_hbm, g_v, acc_v, pk0, pk1, w0, w1,
                sem0, sem1):
    c = lax.axis_index("c")
    s = lax.axis_index("s")

    pltpu.sync_copy(gT_hbm.at[pl.ds(s * N_PAD, N_PAD)], g_v)

    def zb(i, carry):
        acc_v[pl.ds(i * 16, 16)] = jnp.zeros((16,), jnp.float32)
        return carry

    lax.fori_loop(0, N_PAD // 16, zb, 0, unroll=8)

    ebase = c * EDGES_PER_CORE
    pks, ws, sems = (pk0, pk1), (w0, w1), (sem0, sem1)

    def start(k, b, n):
        pltpu.async_copy(pk_hbm.at[pl.ds(ebase + k * CHUNK, n)],
                         pks[b].at[pl.ds(0, n)], sems[b])
        pltpu.async_copy(w_hbm.at[pl.ds(ebase + k * CHUNK, n)],
                         ws[b].at[pl.ds(0, n)], sems[b])

    def wait(b, n):
        pltpu.make_async_copy(pk_hbm.at[pl.ds(0, n)], pks[b].at[pl.ds(0, n)],
                              sems[b]).wait()
        pltpu.make_async_copy(w_hbm.at[pl.ds(0, n)], ws[b].at[pl.ds(0, n)],
                              sems[b]).wait()

    def process(b, n):
        pk_v, w_v = pks[b], ws[b]

        @plsc.parallel_loop(0, n, step=16, unroll=16)
        def inner(j):
            pk = pk_v[pl.ds(j, 16)]
            src = jnp.bitwise_and(pk, jnp.int32(0xFFFF))
            dst = lax.shift_right_logical(pk, 16)
            wv = w_v[pl.ds(j, 16)]
            gv = plsc.load_gather(g_v, [src])
            plsc.addupdate_scatter(acc_v, [dst], gv * wv)

    start(0, 0, CHUNK)

    def outer(g, carry):
        for b in range(2):
            k = g * 2 + b

            @pl.when(k + 1 < FULL_CHUNKS)
            def _():
                start(k + 1, 1 - b, CHUNK)

            wait(b, CHUNK)
            process(b, CHUNK)
        return carry

    lax.fori_loop(0, FULL_CHUNKS // 2, outer, 0)
    start(FULL_CHUNKS, 1, TAIL)
    wait(0, CHUNK)
    process(0, CHUNK)
    wait(1, TAIL)
    process(1, TAIL)
    pltpu.sync_copy(acc_v, out_hbm.at[pl.ds((c * F_HID + s) * N_PAD, N_PAD)])


EBLK = 16384
E_GRID = E_PAD // EBLK


def _pack_body(ei_ref, w_ref, pk_ref, dstp_ref, wp_ref):
    i = pl.program_id(0)
    col = i * EBLK + jax.lax.broadcasted_iota(jnp.int32, (1, EBLK), 1)
    valid = col < E
    src = ei_ref[0:1, :]
    dstv = ei_ref[1:2, :]
    pk_ref[...] = jnp.bitwise_or(src, lax.shift_left(dstv, 16))
    dstp_ref[...] = jnp.where(valid, dstv, 0)
    wp_ref[...] = jnp.where(valid, w_ref[...], 0.0)


def _pack_call(ei, w):
    return pl.pallas_call(
        _pack_body,
        grid=(E_GRID,),
        in_specs=[
            pl.BlockSpec((2, EBLK), lambda i: (0, i)),
            pl.BlockSpec((1, EBLK), lambda i: (0, i)),
        ],
        out_specs=[
            pl.BlockSpec((1, EBLK), lambda i: (0, i)),
            pl.BlockSpec((1, EBLK), lambda i: (0, i)),
            pl.BlockSpec((1, EBLK), lambda i: (0, i)),
        ],
        out_shape=[
            jax.ShapeDtypeStruct((1, E), jnp.int32),
            jax.ShapeDtypeStruct((1, E_PAD), jnp.int32),
            jax.ShapeDtypeStruct((1, E_PAD), jnp.float32),
        ],
    )(ei, w)


def _prep1_body(deg_ref, x_ref, w1_ref, dis_ref, g_ref):
    deg = deg_ref[0:1, :] + deg_ref[1:2, :] + 1.0
    dis = lax.rsqrt(deg)
    dis_ref[...] = dis
    h = lax.dot_general(
        w1_ref[...], x_ref[...], (((0,), (1,)), ((), ())),
        preferred_element_type=jnp.float32,
    )
    g_ref[...] = h * dis


def _prep1_call(deg, xp, W1):
    return pl.pallas_call(
        _prep1_body,
        grid=(N_GRID,),
        in_specs=[
            pl.BlockSpec((2, NBLK), lambda i: (0, i)),
            pl.BlockSpec((NBLK, F_IN), lambda i: (i, 0)),
            pl.BlockSpec((F_IN, F_HID), lambda i: (0, 0)),
        ],
        out_specs=[
            pl.BlockSpec((1, NBLK), lambda i: (0, i)),
            pl.BlockSpec((F_HID, NBLK), lambda i: (0, i)),
        ],
        out_shape=[
            jax.ShapeDtypeStruct((1, N_PAD), jnp.float32),
            jax.ShapeDtypeStruct((F_HID, N_PAD), jnp.float32),
        ],
    )(deg, xp, W1)


def _mid_body(agg_ref, g1_ref, dis_ref, b1_ref, w2_ref, g2_ref):
    agg = agg_ref[0] + agg_ref[1]
    dis = dis_ref[...]
    o1 = dis * (agg + g1_ref[...]) + b1_ref[:, 0:1]
    h1 = jnp.maximum(o1, 0.0)
    h2 = lax.dot_general(
        w2_ref[...], h1, (((0,), (0,)), ((), ())),
        preferred_element_type=jnp.float32,
    )
    g2_ref[...] = h2 * dis


def _mid_call(agg1, g1T, dis, b1b, W2):
    return pl.pallas_call(
        _mid_body,
        grid=(N_GRID,),
        in_specs=[
            pl.BlockSpec((2, F_HID, NBLK), lambda i: (0, 0, i)),
            pl.BlockSpec((F_HID, NBLK), lambda i: (0, i)),
            pl.BlockSpec((1, NBLK), lambda i: (0, i)),
            pl.BlockSpec((F_HID, 128), lambda i: (0, 0)),
            pl.BlockSpec((F_HID, F_HID), lambda i: (0, 0)),
        ],
        out_specs=pl.BlockSpec((F_HID, NBLK), lambda i: (0, i)),
        out_shape=jax.ShapeDtypeStruct((F_HID, N_PAD), jnp.float32),
    )(agg1, g1T, dis, b1b, W2)


def _fin_body(agg_ref, g2_ref, dis_ref, b2_ref, wp_ref, bp_ref, out_ref):
    agg = agg_ref[0] + agg_ref[1]
    dis = dis_ref[...]
    o2 = dis * (agg + g2_ref[...]) + b2_ref[:, 0:1]
    z = jnp.sum(wp_ref[:, 0:1] * o2, axis=0, keepdims=True) + bp_ref[0:1, 0:1]
    out_ref[...] = jax.nn.sigmoid(z)


def _fin_call(agg2, g2T, dis, b2b, wpb, bpb):
    return pl.pallas_call(
        _fin_body,
        grid=(N_GRID,),
        in_specs=[
            pl.BlockSpec((2, F_HID, NBLK), lambda i: (0, 0, i)),
            pl.BlockSpec((F_HID, NBLK), lambda i: (0, i)),
            pl.BlockSpec((1, NBLK), lambda i: (0, i)),
            pl.BlockSpec((F_HID, 128), lambda i: (0, 0)),
            pl.BlockSpec((F_HID, 128), lambda i: (0, 0)),
            pl.BlockSpec((1, 128), lambda i: (0, 0)),
        ],
        out_specs=pl.BlockSpec((1, NBLK), lambda i: (0, i)),
        out_shape=jax.ShapeDtypeStruct((1, N_PAD), jnp.float32),
    )(agg2, g2T, dis, b2b, wpb, bpb)


def kernel(x, edge_index, edge_weight, W1, b1, W2, b2, Wp, bp):
    w = edge_weight

    pk, dstp, wp = _pack_call(edge_index, w.reshape(1, E))
    pk = pk.reshape(-1)
    deg = _deg_kernel(dstp.reshape(-1, 128), wp.reshape(-1, 128))
    dis, g1T = _prep1_call(deg.reshape(2, N_PAD), x, W1)
    agg1 = _agg_kernel(g1T.reshape(-1), pk, w)
    b1b = jnp.tile(b1[:, None], (1, 128))
    g2T = _mid_call(agg1.reshape(2, F_HID, N_PAD), g1T, dis, b1b, W2)
    agg2 = _agg_kernel(g2T.reshape(-1), pk, w)
    b2b = jnp.tile(b2[:, None], (1, 128))
    wpb = jnp.tile(Wp, (1, 128))
    bpb = jnp.tile(bp[:, None], (1, 128))
    outT = _fin_call(agg2.reshape(2, F_HID, N_PAD), g2T, dis, b2b, wpb, bpb)
    return outT[0, :N][:, None]


# speedup vs baseline: 1.0036x; 1.0036x over previous
import functools

import jax
import jax.numpy as jnp
from jax import lax
from jax.experimental import pallas as pl
from jax.experimental.pallas import tpu as pltpu
from jax.experimental.pallas import tpu_sc as plsc

N = 50000
E = 1600000
F_IN = 50
F_HID = 16

N_PAD = 50176
EDGES_PER_CORE = E // 2
CHUNK = 4096
FULL_CHUNKS = EDGES_PER_CORE // CHUNK
TAIL = EDGES_PER_CORE - FULL_CHUNKS * CHUNK
E_PAD = 1605632
DEG_ROWS_PER_TILE = E_PAD // 128 // 32
DEG_CHUNK_ROWS = 8
NCH_D = DEG_ROWS_PER_TILE // DEG_CHUNK_ROWS
N_SLICE = N_PAD // 16

NBLK = 1024
N_GRID = N_PAD // NBLK

_sc_mesh = plsc.VectorSubcoreMesh(core_axis_name="c", subcore_axis_name="s")


@functools.partial(
    pl.kernel,
    out_type=jax.ShapeDtypeStruct((2 * N_PAD,), jnp.float32),
    mesh=_sc_mesh,
    compiler_params=pltpu.CompilerParams(needs_layout_passes=False),
    scratch_types=[
        pltpu.VMEM((DEG_CHUNK_ROWS, 128), jnp.int32),
        pltpu.VMEM((DEG_CHUNK_ROWS, 128), jnp.int32),
        pltpu.VMEM((DEG_CHUNK_ROWS, 128), jnp.float32),
        pltpu.VMEM((DEG_CHUNK_ROWS, 128), jnp.float32),
        pltpu.VMEM((N_SLICE,), jnp.float32),
        pltpu.VMEM_SHARED((N_PAD,), jnp.float32),
        pltpu.SemaphoreType.DMA,
        pltpu.SemaphoreType.DMA,
        pltpu.SemaphoreType.DMA,
    ],
)
def _deg_kernel(dst_hbm, w_hbm, deg_hbm, dst0, dst1, w0, w1, z_v, deg_sh,
                sem0, sem1, sem_sc):
    c = lax.axis_index("c")
    s = lax.axis_index("s")

    def zb(i, carry):
        z_v[pl.ds(i * 16, 16)] = jnp.zeros((16,), jnp.float32)
        return carry

    lax.fori_loop(0, N_SLICE // 16, zb, 0, unroll=8)
    pltpu.sync_copy(z_v, deg_sh.at[pl.ds(s * N_SLICE, N_SLICE)])
    plsc.subcore_barrier()

    row_base = (c * 16 + s) * DEG_ROWS_PER_TILE
    dsts, ws, sems = (dst0, dst1), (w0, w1), (sem0, sem1)

    def start(k, b):
        r0 = row_base + k * DEG_CHUNK_ROWS
        pltpu.async_copy(dst_hbm.at[pl.ds(r0, DEG_CHUNK_ROWS)], dsts[b],
                         sems[b])
        pltpu.async_copy(w_hbm.at[pl.ds(r0, DEG_CHUNK_ROWS)], ws[b],
                         sems[b])

    def wait_in(b):
        pltpu.make_async_copy(dst_hbm.at[pl.ds(0, DEG_CHUNK_ROWS)], dsts[b],
                              sems[b]).wait()
        pltpu.make_async_copy(w_hbm.at[pl.ds(0, DEG_CHUNK_ROWS)], ws[b],
                              sems[b]).wait()

    def process(b):
        dst_v, w_v = dsts[b], ws[b]
        for j in range(DEG_CHUNK_ROWS):
            pltpu.async_copy(w_v.at[j], deg_sh.at[dst_v.at[j]], sem_sc,
                             add=True)
        pltpu.make_async_copy(w_hbm.at[pl.ds(0, DEG_CHUNK_ROWS)], w_v,
                              sem_sc).wait()

    start(0, 0)

    def outer(g, carry):
        for b in range(2):
            k = g * 2 + b

            @pl.when(k + 1 < NCH_D)
            def _():
                start(k + 1, 1 - b)

            wait_in(b)
            process(b)
        return carry

    lax.fori_loop(0, NCH_D // 2, outer, 0)
    wait_in(0)
    process(0)

    plsc.subcore_barrier()
    pltpu.sync_copy(deg_sh.at[pl.ds(s * N_SLICE, N_SLICE)], z_v)
    pltpu.sync_copy(z_v, deg_hbm.at[pl.ds(c * N_PAD + s * N_SLICE, N_SLICE)])


@functools.partial(
    pl.kernel,
    out_type=jax.ShapeDtypeStruct((2 * F_HID * N_PAD,), jnp.float32),
    mesh=_sc_mesh,
    compiler_params=pltpu.CompilerParams(needs_layout_passes=False),
    scratch_types=[
        pltpu.VMEM((N_PAD,), jnp.float32),
        pltpu.VMEM((N_PAD,), jnp.float32),
        pltpu.VMEM((CHUNK,), jnp.int32),
        pltpu.VMEM((CHUNK,), jnp.int32),
        pltpu.VMEM((CHUNK,), jnp.float32),
        pltpu.VMEM((CHUNK,), jnp.float32),
        pltpu.SemaphoreType.DMA,
        pltpu.SemaphoreType.DMA,
    ],
)
def _agg_kernel(gT_hbm, pk_hbm, w_hbm, out_hbm, g_v, acc_v, pk0, pk1, w0, w1,
                sem0, sem1):
    c = lax.axis_index("c")
    s = lax.axis_index("s")

    pltpu.sync_copy(gT_hbm.at[pl.ds(s * N_PAD, N_PAD)], g_v)

    def zb(i, carry):
        acc_v[pl.ds(i * 16, 16)] = jnp.zeros((16,), jnp.float32)
        return carry

    lax.fori_loop(0, N_PAD // 16, zb, 0, unroll=8)

    ebase = c * EDGES_PER_CORE
    pks, ws, sems = (pk0, pk1), (w0, w1), (sem0, sem1)

    def start(k, b, n):
        pltpu.async_copy(pk_hbm.at[pl.ds(ebase + k * CHUNK, n)],
                         pks[b].at[pl.ds(0, n)], sems[b])
        pltpu.async_copy(w_hbm.at[pl.ds(ebase + k * CHUNK, n)],
                         ws[b].at[pl.ds(0, n)], sems[b])

    def wait(b, n):
        pltpu.make_async_copy(pk_hbm.at[pl.ds(0, n)], pks[b].at[pl.ds(0, n)],
                              sems[b]).wait()
        pltpu.make_async_copy(w_hbm.at[pl.ds(0, n)], ws[b].at[pl.ds(0, n)],
                              sems[b]).wait()

    def process(b, n):
        pk_v, w_v = pks[b], ws[b]

        @plsc.parallel_loop(0, n, step=16, unroll=8)
        def inner(j):
            pk = pk_v[pl.ds(j, 16)]
            src = jnp.bitwise_and(pk, jnp.int32(0xFFFF))
            dst = lax.shift_right_logical(pk, 16)
            wv = w_v[pl.ds(j, 16)]
            gv = plsc.load_gather(g_v, [src])
            plsc.addupdate_scatter(acc_v, [dst], gv * wv)

    start(0, 0, CHUNK)

    def outer(g, carry):
        for b in range(2):
            k = g * 2 + b

            @pl.when(k + 1 < FULL_CHUNKS)
            def _():
                start(k + 1, 1 - b, CHUNK)

            wait(b, CHUNK)
            process(b, CHUNK)
        return carry

    lax.fori_loop(0, FULL_CHUNKS // 2, outer, 0)
    start(FULL_CHUNKS, 1, TAIL)
    wait(0, CHUNK)
    process(0, CHUNK)
    wait(1, TAIL)
    process(1, TAIL)
    pltpu.sync_copy(acc_v, out_hbm.at[pl.ds((c * F_HID + s) * N_PAD, N_PAD)])


EBLK = 16384
E_GRID = E_PAD // EBLK


def _pack_body(ei_ref, w_ref, pk_ref, dstp_ref, wp_ref):
    i = pl.program_id(0)
    col = i * EBLK + jax.lax.broadcasted_iota(jnp.int32, (1, EBLK), 1)
    valid = col < E
    src = ei_ref[0:1, :]
    dstv = ei_ref[1:2, :]
    pk_ref[...] = jnp.bitwise_or(src, lax.shift_left(dstv, 16))
    dstp_ref[...] = jnp.where(valid, dstv, 0)
    wp_ref[...] = jnp.where(valid, w_ref[...], 0.0)


def _pack_call(ei, w):
    return pl.pallas_call(
        _pack_body,
        grid=(E_GRID,),
        in_specs=[
            pl.BlockSpec((2, EBLK), lambda i: (0, i)),
            pl.BlockSpec((1, EBLK), lambda i: (0, i)),
        ],
        out_specs=[
            pl.BlockSpec((1, EBLK), lambda i: (0, i)),
            pl.BlockSpec((1, EBLK), lambda i: (0, i)),
            pl.BlockSpec((1, EBLK), lambda i: (0, i)),
        ],
        out_shape=[
            jax.ShapeDtypeStruct((1, E), jnp.int32),
            jax.ShapeDtypeStruct((1, E_PAD), jnp.int32),
            jax.ShapeDtypeStruct((1, E_PAD), jnp.float32),
        ],
    )(ei, w)


def _prep1_body(deg_ref, x_ref, w1_ref, dis_ref, g_ref):
    deg = deg_ref[0:1, :] + deg_ref[1:2, :] + 1.0
    dis = lax.rsqrt(deg)
    dis_ref[...] = dis
    h = lax.dot_general(
        w1_ref[...], x_ref[...], (((0,), (1,)), ((), ())),
        preferred_element_type=jnp.float32,
    )
    g_ref[...] = h * dis


def _prep1_call(deg, xp, W1):
    return pl.pallas_call(
        _prep1_body,
        grid=(N_GRID,),
        in_specs=[
            pl.BlockSpec((2, NBLK), lambda i: (0, i)),
            pl.BlockSpec((NBLK, F_IN), lambda i: (i, 0)),
            pl.BlockSpec((F_IN, F_HID), lambda i: (0, 0)),
        ],
        out_specs=[
            pl.BlockSpec((1, NBLK), lambda i: (0, i)),
            pl.BlockSpec((F_HID, NBLK), lambda i: (0, i)),
        ],
        out_shape=[
            jax.ShapeDtypeStruct((1, N_PAD), jnp.float32),
            jax.ShapeDtypeStruct((F_HID, N_PAD), jnp.float32),
        ],
    )(deg, xp, W1)


def _mid_body(agg_ref, g1_ref, dis_ref, b1_ref, w2_ref, g2_ref):
    agg = agg_ref[0] + agg_ref[1]
    dis = dis_ref[...]
    o1 = dis * (agg + g1_ref[...]) + b1_ref[:, 0:1]
    h1 = jnp.maximum(o1, 0.0)
    h2 = lax.dot_general(
        w2_ref[...], h1, (((0,), (0,)), ((), ())),
        preferred_element_type=jnp.float32,
    )
    g2_ref[...] = h2 * dis


def _mid_call(agg1, g1T, dis, b1b, W2):
    return pl.pallas_call(
        _mid_body,
        grid=(N_GRID,),
        in_specs=[
            pl.BlockSpec((2, F_HID, NBLK), lambda i: (0, 0, i)),
            pl.BlockSpec((F_HID, NBLK), lambda i: (0, i)),
            pl.BlockSpec((1, NBLK), lambda i: (0, i)),
            pl.BlockSpec((F_HID, 128), lambda i: (0, 0)),
            pl.BlockSpec((F_HID, F_HID), lambda i: (0, 0)),
        ],
        out_specs=pl.BlockSpec((F_HID, NBLK), lambda i: (0, i)),
        out_shape=jax.ShapeDtypeStruct((F_HID, N_PAD), jnp.float32),
    )(agg1, g1T, dis, b1b, W2)


def _fin_body(agg_ref, g2_ref, dis_ref, b2_ref, wp_ref, bp_ref, out_ref):
    agg = agg_ref[0] + agg_ref[1]
    dis = dis_ref[...]
    o2 = dis * (agg + g2_ref[...]) + b2_ref[:, 0:1]
    z = jnp.sum(wp_ref[:, 0:1] * o2, axis=0, keepdims=True) + bp_ref[0:1, 0:1]
    out_ref[...] = jax.nn.sigmoid(z)


def _fin_call(agg2, g2T, dis, b2b, wpb, bpb):
    return pl.pallas_call(
        _fin_body,
        grid=(N_GRID,),
        in_specs=[
            pl.BlockSpec((2, F_HID, NBLK), lambda i: (0, 0, i)),
            pl.BlockSpec((F_HID, NBLK), lambda i: (0, i)),
            pl.BlockSpec((1, NBLK), lambda i: (0, i)),
            pl.BlockSpec((F_HID, 128), lambda i: (0, 0)),
            pl.BlockSpec((F_HID, 128), lambda i: (0, 0)),
            pl.BlockSpec((1, 128), lambda i: (0, 0)),
        ],
        out_specs=pl.BlockSpec((1, NBLK), lambda i: (0, i)),
        out_shape=jax.ShapeDtypeStruct((1, N_PAD), jnp.float32),
    )(agg2, g2T, dis, b2b, wpb, bpb)


def kernel(x, edge_index, edge_weight, W1, b1, W2, b2, Wp, bp):
    w = edge_weight

    pk, dstp, wp = _pack_call(edge_index, w.reshape(1, E))
    pk = pk.reshape(-1)
    deg = _deg_kernel(dstp.reshape(-1, 128), wp.reshape(-1, 128))
    dis, g1T = _prep1_call(deg.reshape(2, N_PAD), x, W1)
    agg1 = _agg_kernel(g1T.reshape(-1), pk, w)
    b1b = jnp.tile(b1[:, None], (1, 128))
    g2T = _mid_call(agg1.reshape(2, F_HID, N_PAD), g1T, dis, b1b, W2)
    agg2 = _agg_kernel(g2T.reshape(-1), pk, w)
    b2b = jnp.tile(b2[:, None], (1, 128))
    wpb = jnp.tile(Wp, (1, 128))
    bpb = jnp.tile(bp[:, None], (1, 128))
    outT = _fin_call(agg2.reshape(2, F_HID, N_PAD), g2T, dis, b2b, wpb, bpb)
    return outT[0, :N][:, None]
